# baseline (speedup 1.0000x reference)
import functools

import jax
import jax.numpy as jnp
import numpy as np
from jax import lax
from jax.experimental import pallas as pl
from jax.experimental.pallas import tpu as pltpu
from jax.experimental.pallas import tpu_sc as plsc

_NUM_CLASSES = 51
_PAD_CLASSES = 64
_FEAT = 1024
_N = 16384
_HEAD = (0, 31, 20, 48, 30, 22, 29, 8, 50, 21)

_SC_ROWS = 4096
_TC_ROWS = _N - _SC_ROWS
_SC_CHUNK = 16
_TC_BLOCK = 2048


def _class_weights() -> np.ndarray:
    w = np.zeros((_PAD_CLASSES, 8), dtype=np.float32)
    head = set(_HEAD)
    for k in range(_NUM_CLASSES):
        fg = 1.0 if k != 0 else 0.0
        hm = 1.0 if k in head else 0.0
        w[k, 0] = fg
        w[k, 1] = fg * hm
        w[k, 2] = fg * (1.0 - hm)
    w[:, 3] = np.arange(_PAD_CLASSES, dtype=np.float32)
    return w


def _tc_body(x_ref, lbl_ref, cen_ref, w_ref, out_ref):
    xb = x_ref[...]
    lbl = lbl_ref[0]
    bsz = xb.shape[0]

    cls = jax.lax.broadcasted_iota(jnp.int32, (_PAD_CLASSES, bsz), 0)
    oh = (cls == lbl).astype(jnp.float32)

    cnorm = jnp.sum(cen_ref[...] * cen_ref[...], axis=1, keepdims=True)
    wlane = jax.lax.broadcasted_iota(jnp.int32, (_PAD_CLASSES, 8), 1)
    wall = jnp.where(wlane == 7, cnorm, w_ref[...])
    cols = jax.lax.dot_general(
        oh, wall,
        dimension_numbers=(((0,), (0,)), ((), ())),
        preferred_element_type=jnp.float32,
    )
    w_fg = cols[:, 0:1]
    w_h = cols[:, 1:2]
    w_nh = cols[:, 2:3]
    lblf = cols[:, 3:4]
    csq = cols[:, 7:8]

    p = jax.lax.dot_general(
        xb, cen_ref[...],
        dimension_numbers=(((1,), (1,)), ((), ())),
        preferred_element_type=jnp.float32,
    )
    ones = jnp.ones((_FEAT, 8), jnp.float32)
    xsq = jax.lax.dot_general(
        xb * xb, ones,
        dimension_numbers=(((1,), (0,)), ((), ())),
        preferred_element_type=jnp.float32,
    )[:, 0:1]

    cls2 = jax.lax.broadcasted_iota(jnp.int32, (bsz, _PAD_CLASSES), 1)
    oht = (cls2 == lblf.astype(jnp.int32)).astype(jnp.float32)
    xc = jnp.sum(p * oht, axis=1, keepdims=True)

    d = xsq + csq - 2.0 * xc
    cd = jnp.clip(d, 1e-8, 1e8)

    s1 = jnp.sum(w_fg * cd)
    s2 = jnp.sum(w_h * cd + w_nh * 1e-8)
    s3 = jnp.sum(w_fg)

    rows = jax.lax.broadcasted_iota(jnp.int32, (8, 128), 0)
    blk = jnp.where(rows == 0, s1, jnp.where(rows == 1, s2,
                    jnp.where(rows == 2, s3, 0.0)))
    out_ref[0] = blk


def _tc_partials(x, lbl_tc, cen, w):
    nb = _TC_ROWS // _TC_BLOCK
    return pl.pallas_call(
        _tc_body,
        grid=(nb,),
        in_specs=[
            pl.BlockSpec((_TC_BLOCK, _FEAT), lambda i: (i, 0)),
            pl.BlockSpec((1, 1, _TC_BLOCK), lambda i: (i, 0, 0)),
            pl.BlockSpec((_PAD_CLASSES, _FEAT), lambda i: (0, 0)),
            pl.BlockSpec((_PAD_CLASSES, 8), lambda i: (0, 0)),
        ],
        out_specs=pl.BlockSpec((1, 8, 128), lambda i: (i, 0, 0)),
        out_shape=jax.ShapeDtypeStruct((nb, 8, 128), jnp.float32),
        compiler_params=pltpu.CompilerParams(
            dimension_semantics=("arbitrary",),
        ),
    )(x, lbl_tc, cen, w)


_COL_UNROLL = 8


def _sc_partials(x, lbl, cen):
    info = plsc.get_sparse_core_info()
    nw = info.num_cores * info.num_subcores
    rpw = _SC_ROWS // nw
    nchunks = rpw // _SC_CHUNK
    mesh = plsc.VectorSubcoreMesh(core_axis_name="c", subcore_axis_name="s")

    @functools.partial(
        pl.kernel,
        out_type=jax.ShapeDtypeStruct((nw, 3, 16), jnp.float32),
        mesh=mesh,
        scratch_types=[
            pltpu.VMEM((rpw,), jnp.int32),
            pltpu.VMEM((_SC_CHUNK, _FEAT), jnp.float32),
            pltpu.VMEM((_PAD_CLASSES, _FEAT), jnp.float32),
            pltpu.VMEM((3, 16), jnp.float32),
            pltpu.SemaphoreType.DMA,
        ],
        compiler_params=pltpu.CompilerParams(use_tc_tiling_on_sc=False, needs_layout_passes=False),
    )
    def body(x_hbm, lbl_hbm, cen_hbm, out_hbm, lbl_v, x_v, cen_v, stage_v, sem):
        wid = lax.axis_index("s") * info.num_cores + lax.axis_index("c")
        base = _TC_ROWS + wid * rpw
        pltpu.async_copy(cen_hbm, cen_v, sem)
        pltpu.sync_copy(lbl_hbm.at[pl.ds(base, rpw)], lbl_v)
        pltpu.make_async_copy(cen_hbm, cen_v, sem).wait()

        zero16 = jnp.zeros((16,), jnp.float32)
        one16 = jnp.ones((16,), jnp.float32)
        eps16 = jnp.full((16,), 1e-8, jnp.float32)
        lane = lax.broadcasted_iota(jnp.int32, (16,), 0)

        def chunk_body(t, carry):
            s1, s2, s3 = carry
            row0 = base + t * _SC_CHUNK
            pltpu.async_copy(x_hbm.at[pl.ds(row0, _SC_CHUNK)], x_v, sem).wait()
            lbl16 = lbl_v[pl.ds(t * _SC_CHUNK, 16)]

            def col_body(c, d16):
                for u in range(_COL_UNROLL):
                    col = jnp.full((16,), c * _COL_UNROLL + u, jnp.int32)
                    xg = plsc.load_gather(x_v, [lane, col])
                    cg = plsc.load_gather(cen_v, [lbl16, col])
                    df = xg - cg
                    d16 = d16 + df * df
                return d16

            d16 = lax.fori_loop(0, _FEAT // _COL_UNROLL, col_body, zero16)

            fg = lbl16 != 0
            hd = lbl16 == _HEAD[0]
            for h in _HEAD[1:]:
                hd = hd | (lbl16 == h)
            cd = jnp.minimum(jnp.maximum(d16, 1e-8), 1e8)
            s1 = s1 + jnp.where(fg, cd, zero16)
            s2 = s2 + jnp.where(fg, jnp.where(hd, cd, eps16), zero16)
            s3 = s3 + jnp.where(fg, one16, zero16)
            return s1, s2, s3

        s1, s2, s3 = lax.fori_loop(
            0, nchunks, chunk_body, (zero16, zero16, zero16)
        )
        stage_v[0] = s1
        stage_v[1] = s2
        stage_v[2] = s3
        pltpu.sync_copy(stage_v, out_hbm.at[wid])

    return body(x, lbl, cen)


@jax.jit
def _run(x, cut_labels, centers):
    cen = jnp.zeros((_PAD_CLASSES, _FEAT), jnp.float32).at[:_NUM_CLASSES].set(centers)
    lbl32 = cut_labels.astype(jnp.int32)
    lbl_tc = lbl32[:_TC_ROWS].reshape(_TC_ROWS // _TC_BLOCK, 1, _TC_BLOCK)
    w = jnp.asarray(_class_weights())

    tc = _tc_partials(x, lbl_tc, cen, w)
    sc = _sc_partials(x, lbl32, cen)

    s1 = jnp.sum(tc[:, 0, 0]) + jnp.sum(sc[:, 0, :])
    s2 = jnp.sum(tc[:, 1, 0]) + jnp.sum(sc[:, 1, :])
    cnt = jnp.maximum(jnp.sum(tc[:, 2, 0]) + jnp.sum(sc[:, 2, :]), 1.0)
    r1 = s1 / cnt
    r2 = s2 / cnt
    r1 = jnp.where(jnp.isnan(r1), 0.0, r1)
    r2 = jnp.where(jnp.isnan(r2), 0.0, r2)
    return r1, r2


def kernel(x, cut_labels, logits, labels, centers):
    del logits, labels
    return _run(x, cut_labels, centers)


# speedup vs baseline: 1.2134x; 1.2134x over previous
import functools

import jax
import jax.numpy as jnp
import numpy as np
from jax import lax
from jax.experimental import pallas as pl
from jax.experimental.pallas import tpu as pltpu
from jax.experimental.pallas import tpu_sc as plsc

_NUM_CLASSES = 51
_PAD_CLASSES = 64
_FEAT = 1024
_N = 16384
_HEAD = (0, 31, 20, 48, 30, 22, 29, 8, 50, 21)

_SC_ROWS = 4096
_TC_ROWS = _N - _SC_ROWS
_SC_CHUNK = 16
_TC_BLOCK = 2048


def _class_weights() -> np.ndarray:
    w = np.zeros((_PAD_CLASSES, 8), dtype=np.float32)
    head = set(_HEAD)
    for k in range(_NUM_CLASSES):
        fg = 1.0 if k != 0 else 0.0
        hm = 1.0 if k in head else 0.0
        w[k, 0] = fg
        w[k, 1] = fg * hm
        w[k, 2] = fg * (1.0 - hm)
    w[:, 3] = np.arange(_PAD_CLASSES, dtype=np.float32)
    return w


def _tc_body(x_ref, lbl_ref, cen_ref, w_ref, out_ref):
    xb = x_ref[...]
    lbl = lbl_ref[0]
    bsz = xb.shape[0]

    cls = jax.lax.broadcasted_iota(jnp.int32, (_PAD_CLASSES, bsz), 0)
    oh = (cls == lbl).astype(jnp.float32)

    cnorm = jnp.sum(cen_ref[...] * cen_ref[...], axis=1, keepdims=True)
    wlane = jax.lax.broadcasted_iota(jnp.int32, (_PAD_CLASSES, 8), 1)
    wall = jnp.where(wlane == 7, cnorm, w_ref[...])
    cols = jax.lax.dot_general(
        oh, wall,
        dimension_numbers=(((0,), (0,)), ((), ())),
        preferred_element_type=jnp.float32,
    )
    w_fg = cols[:, 0:1]
    w_h = cols[:, 1:2]
    w_nh = cols[:, 2:3]
    lblf = cols[:, 3:4]
    csq = cols[:, 7:8]

    p = jax.lax.dot_general(
        xb, cen_ref[...],
        dimension_numbers=(((1,), (1,)), ((), ())),
        preferred_element_type=jnp.float32,
    )
    ones = jnp.ones((_FEAT, 8), jnp.float32)
    xsq = jax.lax.dot_general(
        xb * xb, ones,
        dimension_numbers=(((1,), (0,)), ((), ())),
        preferred_element_type=jnp.float32,
    )[:, 0:1]

    cls2 = jax.lax.broadcasted_iota(jnp.int32, (bsz, _PAD_CLASSES), 1)
    oht = (cls2 == lblf.astype(jnp.int32)).astype(jnp.float32)
    xc = jnp.sum(p * oht, axis=1, keepdims=True)

    d = xsq + csq - 2.0 * xc
    cd = jnp.clip(d, 1e-8, 1e8)

    s1 = jnp.sum(w_fg * cd)
    s2 = jnp.sum(w_h * cd + w_nh * 1e-8)
    s3 = jnp.sum(w_fg)

    rows = jax.lax.broadcasted_iota(jnp.int32, (8, 128), 0)
    blk = jnp.where(rows == 0, s1, jnp.where(rows == 1, s2,
                    jnp.where(rows == 2, s3, 0.0)))
    out_ref[0] = blk


def _tc_partials(x, lbl_tc, cen, w):
    nb = _TC_ROWS // _TC_BLOCK
    return pl.pallas_call(
        _tc_body,
        grid=(nb,),
        in_specs=[
            pl.BlockSpec((_TC_BLOCK, _FEAT), lambda i: (i, 0)),
            pl.BlockSpec((1, 1, _TC_BLOCK), lambda i: (i, 0, 0)),
            pl.BlockSpec((_PAD_CLASSES, _FEAT), lambda i: (0, 0)),
            pl.BlockSpec((_PAD_CLASSES, 8), lambda i: (0, 0)),
        ],
        out_specs=pl.BlockSpec((1, 8, 128), lambda i: (i, 0, 0)),
        out_shape=jax.ShapeDtypeStruct((nb, 8, 128), jnp.float32),
        compiler_params=pltpu.CompilerParams(
            dimension_semantics=("arbitrary",),
        ),
    )(x, lbl_tc, cen, w)


_COL_UNROLL = 8


def _sc_partials(x, lbl, cen):
    info = plsc.get_sparse_core_info()
    nw = info.num_cores * info.num_subcores
    rpw = _SC_ROWS // nw
    nchunks = rpw // _SC_CHUNK
    mesh = plsc.VectorSubcoreMesh(core_axis_name="c", subcore_axis_name="s")

    @functools.partial(
        pl.kernel,
        out_type=jax.ShapeDtypeStruct((nw, 3, 16), jnp.float32),
        mesh=mesh,
        scratch_types=[
            pltpu.VMEM((rpw,), jnp.int32),
            pltpu.VMEM((_SC_CHUNK, _FEAT), jnp.float32),
            pltpu.VMEM((_PAD_CLASSES, _FEAT), jnp.float32),
            pltpu.VMEM((3, 16), jnp.float32),
            pltpu.SemaphoreType.DMA,
        ],
        compiler_params=pltpu.CompilerParams(needs_layout_passes=False),
    )
    def body(x_hbm, lbl_hbm, cen_hbm, out_hbm, lbl_v, x_v, cen_v, stage_v, sem):
        wid = lax.axis_index("s") * info.num_cores + lax.axis_index("c")
        base = _TC_ROWS + wid * rpw
        pltpu.async_copy(cen_hbm, cen_v, sem)
        pltpu.sync_copy(lbl_hbm.at[pl.ds(base, rpw)], lbl_v)
        pltpu.make_async_copy(cen_hbm, cen_v, sem).wait()

        zero16 = jnp.zeros((16,), jnp.float32)
        one16 = jnp.ones((16,), jnp.float32)
        eps16 = jnp.full((16,), 1e-8, jnp.float32)
        lane = lax.broadcasted_iota(jnp.int32, (16,), 0)

        def chunk_body(t, carry):
            s1, s2, s3 = carry
            row0 = base + t * _SC_CHUNK
            pltpu.async_copy(x_hbm.at[pl.ds(row0, _SC_CHUNK)], x_v, sem).wait()
            lbl16 = lbl_v[pl.ds(t * _SC_CHUNK, 16)]

            def col_body(c, d16):
                for u in range(_COL_UNROLL):
                    col = jnp.full((16,), c * _COL_UNROLL + u, jnp.int32)
                    xg = plsc.load_gather(x_v, [lane, col])
                    cg = plsc.load_gather(cen_v, [lbl16, col])
                    df = xg - cg
                    d16 = d16 + df * df
                return d16

            d16 = lax.fori_loop(0, _FEAT // _COL_UNROLL, col_body, zero16)

            fg = lbl16 != 0
            hd = lbl16 == _HEAD[0]
            for h in _HEAD[1:]:
                hd = hd | (lbl16 == h)
            cd = jnp.minimum(jnp.maximum(d16, 1e-8), 1e8)
            s1 = s1 + jnp.where(fg, cd, zero16)
            s2 = s2 + jnp.where(fg, jnp.where(hd, cd, eps16), zero16)
            s3 = s3 + jnp.where(fg, one16, zero16)
            return s1, s2, s3

        s1, s2, s3 = lax.fori_loop(
            0, nchunks, chunk_body, (zero16, zero16, zero16)
        )
        stage_v[0] = s1
        stage_v[1] = s2
        stage_v[2] = s3
        pltpu.sync_copy(stage_v, out_hbm.at[wid])

    return body(x, lbl, cen)


@jax.jit
def _run(x, cut_labels, centers):
    cen = jnp.zeros((_PAD_CLASSES, _FEAT), jnp.float32).at[:_NUM_CLASSES].set(centers)
    lbl32 = cut_labels.astype(jnp.int32)
    lbl_tc = lbl32[:_TC_ROWS].reshape(_TC_ROWS // _TC_BLOCK, 1, _TC_BLOCK)
    w = jnp.asarray(_class_weights())

    tc = _tc_partials(x, lbl_tc, cen, w)
    sc = _sc_partials(x, lbl32, cen)

    s1 = jnp.sum(tc[:, 0, 0]) + jnp.sum(sc[:, 0, :])
    s2 = jnp.sum(tc[:, 1, 0]) + jnp.sum(sc[:, 1, :])
    cnt = jnp.maximum(jnp.sum(tc[:, 2, 0]) + jnp.sum(sc[:, 2, :]), 1.0)
    r1 = s1 / cnt
    r2 = s2 / cnt
    r1 = jnp.where(jnp.isnan(r1), 0.0, r1)
    r2 = jnp.where(jnp.isnan(r2), 0.0, r2)
    return r1, r2


def kernel(x, cut_labels, logits, labels, centers):
    del logits, labels
    return _run(x, cut_labels, centers)


# speedup vs baseline: 3.2302x; 2.6621x over previous
import functools

import jax
import jax.numpy as jnp
import numpy as np
from jax import lax
from jax.experimental import pallas as pl
from jax.experimental.pallas import tpu as pltpu
from jax.experimental.pallas import tpu_sc as plsc

_NUM_CLASSES = 51
_PAD_CLASSES = 64
_FEAT = 1024
_N = 16384
_HEAD = (0, 31, 20, 48, 30, 22, 29, 8, 50, 21)

_SC_ROWS = 4096
_TC_ROWS = _N - _SC_ROWS
_SC_CHUNK = 16
_TC_BLOCK = 2048


def _class_weights() -> np.ndarray:
    w = np.zeros((_PAD_CLASSES, 8), dtype=np.float32)
    head = set(_HEAD)
    for k in range(_NUM_CLASSES):
        fg = 1.0 if k != 0 else 0.0
        hm = 1.0 if k in head else 0.0
        w[k, 0] = fg
        w[k, 1] = fg * hm
        w[k, 2] = fg * (1.0 - hm)
    w[:, 3] = np.arange(_PAD_CLASSES, dtype=np.float32)
    return w


def _tc_body(x_ref, lbl_ref, cen_ref, w_ref, out_ref):
    xb = x_ref[...]
    lbl = lbl_ref[0]
    bsz = xb.shape[0]

    cls = jax.lax.broadcasted_iota(jnp.int32, (_PAD_CLASSES, bsz), 0)
    oh = (cls == lbl).astype(jnp.float32)

    cnorm = jnp.sum(cen_ref[...] * cen_ref[...], axis=1, keepdims=True)
    wlane = jax.lax.broadcasted_iota(jnp.int32, (_PAD_CLASSES, 8), 1)
    wall = jnp.where(wlane == 7, cnorm, w_ref[...])
    cols = jax.lax.dot_general(
        oh, wall,
        dimension_numbers=(((0,), (0,)), ((), ())),
        preferred_element_type=jnp.float32,
    )
    w_fg = cols[:, 0:1]
    w_h = cols[:, 1:2]
    w_nh = cols[:, 2:3]
    lblf = cols[:, 3:4]
    csq = cols[:, 7:8]

    p = jax.lax.dot_general(
        xb, cen_ref[...],
        dimension_numbers=(((1,), (1,)), ((), ())),
        preferred_element_type=jnp.float32,
    )
    ones = jnp.ones((_FEAT, 8), jnp.float32)
    xsq = jax.lax.dot_general(
        xb * xb, ones,
        dimension_numbers=(((1,), (0,)), ((), ())),
        preferred_element_type=jnp.float32,
    )[:, 0:1]

    cls2 = jax.lax.broadcasted_iota(jnp.int32, (bsz, _PAD_CLASSES), 1)
    oht = (cls2 == lblf.astype(jnp.int32)).astype(jnp.float32)
    xc = jnp.sum(p * oht, axis=1, keepdims=True)

    d = xsq + csq - 2.0 * xc
    cd = jnp.clip(d, 1e-8, 1e8)

    s1 = jnp.sum(w_fg * cd)
    s2 = jnp.sum(w_h * cd + w_nh * 1e-8)
    s3 = jnp.sum(w_fg)

    rows = jax.lax.broadcasted_iota(jnp.int32, (8, 128), 0)
    blk = jnp.where(rows == 0, s1, jnp.where(rows == 1, s2,
                    jnp.where(rows == 2, s3, 0.0)))
    out_ref[0] = blk


def _tc_partials(x, lbl_tc, cen, w):
    nb = _TC_ROWS // _TC_BLOCK
    return pl.pallas_call(
        _tc_body,
        grid=(nb,),
        in_specs=[
            pl.BlockSpec((_TC_BLOCK, _FEAT), lambda i: (i, 0)),
            pl.BlockSpec((1, 1, _TC_BLOCK), lambda i: (i, 0, 0)),
            pl.BlockSpec((_PAD_CLASSES, _FEAT), lambda i: (0, 0)),
            pl.BlockSpec((_PAD_CLASSES, 8), lambda i: (0, 0)),
        ],
        out_specs=pl.BlockSpec((1, 8, 128), lambda i: (i, 0, 0)),
        out_shape=jax.ShapeDtypeStruct((nb, 8, 128), jnp.float32),
        compiler_params=pltpu.CompilerParams(
            dimension_semantics=("arbitrary",),
        ),
    )(x, lbl_tc, cen, w)


_COL_UNROLL = 8


def _sc_partials(x, lbl, cen):
    info = plsc.get_sparse_core_info()
    nw = info.num_cores * info.num_subcores
    rpw = _SC_ROWS // nw
    nchunks = rpw // _SC_CHUNK
    mesh = plsc.VectorSubcoreMesh(core_axis_name="c", subcore_axis_name="s")

    @functools.partial(
        pl.kernel,
        out_type=jax.ShapeDtypeStruct((nw, 3, 16), jnp.float32),
        mesh=mesh,
        scratch_types=[
            pltpu.VMEM((rpw,), jnp.int32),
            pltpu.VMEM((_SC_CHUNK, _FEAT), jnp.float32),
            pltpu.VMEM((_SC_CHUNK, _FEAT), jnp.float32),
            pltpu.VMEM((_SC_CHUNK, 16), jnp.float32),
            pltpu.VMEM((3, 16), jnp.float32),
            pltpu.SemaphoreType.DMA,
        ],
        compiler_params=pltpu.CompilerParams(needs_layout_passes=False),
    )
    def body(x_hbm, lbl_hbm, cen_hbm, out_hbm,
             lbl_v, x_v, c_v, racc_v, stage_v, sem):
        wid = lax.axis_index("s") * info.num_cores + lax.axis_index("c")
        base = _TC_ROWS + wid * rpw
        pltpu.sync_copy(lbl_hbm.at[pl.ds(base, rpw)], lbl_v)

        zero16 = jnp.zeros((16,), jnp.float32)
        one16 = jnp.ones((16,), jnp.float32)
        eps16 = jnp.full((16,), 1e-8, jnp.float32)
        lane = lax.broadcasted_iota(jnp.int32, (16,), 0)

        def chunk_body(t, carry):
            s1, s2, s3 = carry
            row0 = base + t * _SC_CHUNK
            xd = pltpu.async_copy(x_hbm.at[pl.ds(row0, _SC_CHUNK)], x_v, sem)
            cd = pltpu.async_copy(
                cen_hbm.at[lbl_v.at[pl.ds(t * _SC_CHUNK, _SC_CHUNK)]], c_v, sem
            )
            xd.wait()
            cd.wait()
            lbl16 = lbl_v[pl.ds(t * _SC_CHUNK, 16)]

            def row_body(j, acc0):
                def k_body(k, acc):
                    b = k * (16 * _COL_UNROLL)
                    for u in range(_COL_UNROLL):
                        xv = x_v[j, pl.ds(b + u * 16, 16)]
                        cv = c_v[j, pl.ds(b + u * 16, 16)]
                        df = xv - cv
                        acc = acc + df * df
                    return acc
                acc = lax.fori_loop(
                    0, _FEAT // (16 * _COL_UNROLL), k_body, zero16)
                racc_v[j] = acc
                return acc0

            lax.fori_loop(0, _SC_CHUNK, row_body, zero16)

            d16 = zero16
            for u in range(16):
                d16 = d16 + plsc.load_gather(
                    racc_v, [lane, jnp.full((16,), u, jnp.int32)])

            fg = lbl16 != 0
            hd = lbl16 == _HEAD[0]
            for h in _HEAD[1:]:
                hd = hd | (lbl16 == h)
            cd = jnp.minimum(jnp.maximum(d16, 1e-8), 1e8)
            s1 = s1 + jnp.where(fg, cd, zero16)
            s2 = s2 + jnp.where(fg, jnp.where(hd, cd, eps16), zero16)
            s3 = s3 + jnp.where(fg, one16, zero16)
            return s1, s2, s3

        s1, s2, s3 = lax.fori_loop(
            0, nchunks, chunk_body, (zero16, zero16, zero16)
        )
        stage_v[0] = s1
        stage_v[1] = s2
        stage_v[2] = s3
        pltpu.sync_copy(stage_v, out_hbm.at[wid])

    return body(x, lbl, cen)


@jax.jit
def _run(x, cut_labels, centers):
    cen = jnp.zeros((_PAD_CLASSES, _FEAT), jnp.float32).at[:_NUM_CLASSES].set(centers)
    lbl32 = cut_labels.astype(jnp.int32)
    lbl_tc = lbl32[:_TC_ROWS].reshape(_TC_ROWS // _TC_BLOCK, 1, _TC_BLOCK)
    w = jnp.asarray(_class_weights())

    tc = _tc_partials(x, lbl_tc, cen, w)
    sc = _sc_partials(x, lbl32, cen)

    s1 = jnp.sum(tc[:, 0, 0]) + jnp.sum(sc[:, 0, :])
    s2 = jnp.sum(tc[:, 1, 0]) + jnp.sum(sc[:, 1, :])
    cnt = jnp.maximum(jnp.sum(tc[:, 2, 0]) + jnp.sum(sc[:, 2, :]), 1.0)
    r1 = s1 / cnt
    r2 = s2 / cnt
    r1 = jnp.where(jnp.isnan(r1), 0.0, r1)
    r2 = jnp.where(jnp.isnan(r2), 0.0, r2)
    return r1, r2


def kernel(x, cut_labels, logits, labels, centers):
    del logits, labels
    return _run(x, cut_labels, centers)
